# baseline (speedup 1.0000x reference)
import functools

import jax
import jax.numpy as jnp
from jax import lax
from jax.experimental import pallas as pl
from jax.experimental.pallas import tpu as pltpu
from jax.experimental.pallas import tpu_sc as plsc

EMBED = 64
BATCH = 16384
HIDDEN = 128

NC = 2
NS = 16
NW = NC * NS
B_PER_W = BATCH // NW
PASS = 256
N_PASS = B_PER_W // PASS


B_PER_T = BATCH // NS


STREAM_G = 10
N_STREAM = STREAM_G * 16
N_DIRECT = PASS - N_STREAM


def _gather_body(table, ids, out, idx, buf0, buf1, sem, dsem):
    tid = lax.axis_index("s")
    base = tid * B_PER_T
    pltpu.sync_copy(ids.at[pl.ds(base, B_PER_T)], idx)

    n_pass = B_PER_T // PASS
    bufs = (buf0, buf1)
    for p in range(n_pass):
        buf = bufs[p % 2]

        def issue(g, _, p=p, buf=buf):
            off = p * PASS + g * 16
            v = idx[pl.ds(off, 16)]
            for j in range(16):
                src = table.at[pl.ds(v[j], 1)]
                @pl.when(g < STREAM_G)
                def _():
                    pltpu.async_copy(src, buf.at[pl.ds(g * 16 + j, 1)], sem)

                @pl.when(g >= STREAM_G)
                def _():
                    pltpu.async_copy(src, out.at[pl.ds(base + off + j, 1)],
                                     dsem)
            return ()

        lax.fori_loop(0, PASS // 16, issue, ())
        pltpu.make_async_copy(table.at[pl.ds(0, N_STREAM)],
                              buf.at[pl.ds(0, N_STREAM)], sem).wait()
        pltpu.sync_copy(buf.at[pl.ds(0, N_STREAM)],
                        out.at[pl.ds(base + p * PASS, N_STREAM)])
    pltpu.make_async_copy(table.at[pl.ds(0, N_DIRECT * n_pass)],
                          out.at[pl.ds(base, N_DIRECT * n_pass)], dsem).wait()


def _make_gather(name):
    return functools.partial(
        pl.kernel,
        out_type=jax.ShapeDtypeStruct((BATCH, EMBED), jnp.float32),
        mesh=plsc.VectorSubcoreMesh(
            core_axis_name="c", subcore_axis_name="s", num_cores=1
        ),
        scratch_types=[
            pltpu.VMEM((B_PER_T,), jnp.int32),
            pltpu.VMEM((PASS, EMBED), jnp.float32),
            pltpu.VMEM((PASS, EMBED), jnp.float32),
            pltpu.SemaphoreType.DMA,
            pltpu.SemaphoreType.DMA,
        ],
        name=name,
    )(_gather_body)


_gather_u = _make_gather("gather_user")
_gather_m = _make_gather("gather_movie")


BS = 2048


def _mlp_body(u_ref, m_ref, w1u_ref, w1m_ref, b1_ref, w2_ref, b2_ref, out_ref):
    h = jnp.dot(u_ref[...], w1u_ref[...], preferred_element_type=jnp.float32)
    h = h + jnp.dot(m_ref[...], w1m_ref[...], preferred_element_type=jnp.float32)
    h = jnp.maximum(h + b1_ref[...], 0.0)
    out_ref[...] = jnp.sum(h * w2_ref[...], axis=1) + b2_ref[0, 0]


def _mlp(u, m, w1u, w1m, b1, w2, b2):
    grid = (BATCH // BS,)
    return pl.pallas_call(
        _mlp_body,
        grid=grid,
        in_specs=[
            pl.BlockSpec((BS, EMBED), lambda i: (i, 0)),
            pl.BlockSpec((BS, EMBED), lambda i: (i, 0)),
            pl.BlockSpec((EMBED, HIDDEN), lambda i: (0, 0)),
            pl.BlockSpec((EMBED, HIDDEN), lambda i: (0, 0)),
            pl.BlockSpec((1, HIDDEN), lambda i: (0, 0)),
            pl.BlockSpec((1, HIDDEN), lambda i: (0, 0)),
            pl.BlockSpec(memory_space=pltpu.SMEM),
        ],
        out_specs=pl.BlockSpec((BS,), lambda i: (i,)),
        out_shape=jax.ShapeDtypeStruct((BATCH,), jnp.float32),
    )(u, m, w1u, w1m, b1, w2, b2)


def kernel(user_ids, movie_ids, user_table, movie_table, W1, b1, W2, b2):
    uids = user_ids.astype(jnp.int32)
    mids = movie_ids.astype(jnp.int32)
    u = _gather_u(user_table, uids)
    m = _gather_m(movie_table, mids)
    w1t = W1.T
    out = _mlp(u, m, w1t[:EMBED], w1t[EMBED:], b1.reshape(1, HIDDEN),
               W2.reshape(1, HIDDEN), b2.reshape(1, 1))
    return out


# speedup vs baseline: 1.1139x; 1.1139x over previous
import functools

import jax
import jax.numpy as jnp
from jax import lax
from jax.experimental import pallas as pl
from jax.experimental.pallas import tpu as pltpu
from jax.experimental.pallas import tpu_sc as plsc

EMBED = 64
BATCH = 16384
HIDDEN = 128

NC = 2
NS = 16
NW = NC * NS
B_PER_W = BATCH // NW
PASS = 256
N_PASS = B_PER_W // PASS


B_PER_T = BATCH // NS


def _gather_body(table, ids, out, idx, buf0, buf1, sem):
    tid = lax.axis_index("s")
    base = tid * B_PER_T
    pltpu.sync_copy(ids.at[pl.ds(base, B_PER_T)], idx)

    bufs = (buf0, buf1)
    for p in range(B_PER_T // PASS):
        buf = bufs[p % 2]

        def issue(g, _, p=p, buf=buf):
            off = p * PASS + g * 16
            v = idx[pl.ds(off, 16)]
            dst = g * 16
            for j in range(16):
                pltpu.async_copy(table.at[pl.ds(v[j], 1)],
                                 buf.at[pl.ds(dst + j, 1)], sem)
            return ()

        lax.fori_loop(0, PASS // 16, issue, ())
        pltpu.make_async_copy(table.at[pl.ds(0, PASS)], buf, sem).wait()
        pltpu.sync_copy(buf, out.at[pl.ds(base + p * PASS, PASS)])


def _make_gather(name):
    return functools.partial(
        pl.kernel,
        out_type=jax.ShapeDtypeStruct((BATCH, EMBED), jnp.float32),
        mesh=plsc.VectorSubcoreMesh(
            core_axis_name="c", subcore_axis_name="s", num_cores=1
        ),
        scratch_types=[
            pltpu.VMEM((B_PER_T,), jnp.int32),
            pltpu.VMEM((PASS, EMBED), jnp.float32),
            pltpu.VMEM((PASS, EMBED), jnp.float32),
            pltpu.SemaphoreType.DMA,
        ],
        name=name,
        cost_estimate=pl.CostEstimate(
            flops=0,
            transcendentals=0,
            bytes_accessed=600_000_000,
        ),
    )(_gather_body)


_gather_u = _make_gather("gather_user")
_gather_m = _make_gather("gather_movie")


BS = 2048


def _mlp_body(u_ref, m_ref, w1u_ref, w1m_ref, b1_ref, w2_ref, b2_ref, out_ref):
    h = jnp.dot(u_ref[...], w1u_ref[...], preferred_element_type=jnp.float32)
    h = h + jnp.dot(m_ref[...], w1m_ref[...], preferred_element_type=jnp.float32)
    h = jnp.maximum(h + b1_ref[...], 0.0)
    out_ref[...] = jnp.sum(h * w2_ref[...], axis=1) + b2_ref[0, 0]


def _mlp(u, m, w1u, w1m, b1, w2, b2):
    grid = (BATCH // BS,)
    return pl.pallas_call(
        _mlp_body,
        grid=grid,
        in_specs=[
            pl.BlockSpec((BS, EMBED), lambda i: (i, 0)),
            pl.BlockSpec((BS, EMBED), lambda i: (i, 0)),
            pl.BlockSpec((EMBED, HIDDEN), lambda i: (0, 0)),
            pl.BlockSpec((EMBED, HIDDEN), lambda i: (0, 0)),
            pl.BlockSpec((1, HIDDEN), lambda i: (0, 0)),
            pl.BlockSpec((1, HIDDEN), lambda i: (0, 0)),
            pl.BlockSpec(memory_space=pltpu.SMEM),
        ],
        out_specs=pl.BlockSpec((BS,), lambda i: (i,)),
        out_shape=jax.ShapeDtypeStruct((BATCH,), jnp.float32),
    )(u, m, w1u, w1m, b1, w2, b2)


def kernel(user_ids, movie_ids, user_table, movie_table, W1, b1, W2, b2):
    uids = user_ids.astype(jnp.int32)
    mids = movie_ids.astype(jnp.int32)
    u = _gather_u(user_table, uids)
    m = _gather_m(movie_table, mids)
    w1t = W1.T
    out = _mlp(u, m, w1t[:EMBED], w1t[EMBED:], b1.reshape(1, HIDDEN),
               W2.reshape(1, HIDDEN), b2.reshape(1, 1))
    return out


# speedup vs baseline: 1.1219x; 1.0072x over previous
import functools

import jax
import jax.numpy as jnp
from jax import lax
from jax.experimental import pallas as pl
from jax.experimental.pallas import tpu as pltpu
from jax.experimental.pallas import tpu_sc as plsc

EMBED = 64
BATCH = 16384
HIDDEN = 128

NC = 2
NS = 16
NW = NC * NS
B_PER_W = BATCH // NW
PASS = 256
N_PASS = B_PER_W // PASS


B_PER_T = BATCH // NS


def _gather_body(table, ids, out, idx, buf0, buf1, sem0, sem1):
    tid = lax.axis_index("s")
    base = tid * B_PER_T
    pltpu.sync_copy(ids.at[pl.ds(base, B_PER_T)], idx)
    bufs = (buf0, buf1)
    sems = (sem0, sem1)
    n_pass = B_PER_T // PASS

    def issue(p):
        buf = bufs[p % 2]
        sem = sems[p % 2]

        def body(g, _, p=p, buf=buf, sem=sem):
            off = p * PASS + g * 16
            v = idx[pl.ds(off, 16)]
            dst = g * 16
            for j in range(16):
                pltpu.async_copy(table.at[pl.ds(v[j], 1)],
                                 buf.at[pl.ds(dst + j, 1)], sem)
            return ()

        lax.fori_loop(0, PASS // 16, body, ())

    def drain_wb(p):
        buf = bufs[p % 2]
        sem = sems[p % 2]
        pltpu.make_async_copy(table.at[pl.ds(0, PASS)], buf, sem).wait()
        pltpu.sync_copy(buf, out.at[pl.ds(base + p * PASS, PASS)])

    issue(0)
    for p in range(1, n_pass):
        issue(p)
        drain_wb(p - 1)
    drain_wb(n_pass - 1)


def _make_gather(name):
    return functools.partial(
        pl.kernel,
        out_type=jax.ShapeDtypeStruct((BATCH, EMBED), jnp.float32),
        mesh=plsc.VectorSubcoreMesh(
            core_axis_name="c", subcore_axis_name="s", num_cores=1
        ),
        scratch_types=[
            pltpu.VMEM((B_PER_T,), jnp.int32),
            pltpu.VMEM((PASS, EMBED), jnp.float32),
            pltpu.VMEM((PASS, EMBED), jnp.float32),
            pltpu.SemaphoreType.DMA,
            pltpu.SemaphoreType.DMA,
        ],
        name=name,
    )(_gather_body)


_gather_u = _make_gather("gather_user")
_gather_m = _make_gather("gather_movie")


BS = 2048


def _mlp_body(u_ref, m_ref, w1u_ref, w1m_ref, b1_ref, w2_ref, b2_ref, out_ref):
    u = u_ref[...]
    m = m_ref[...]
    h = jnp.dot(u, w1u_ref[...], preferred_element_type=jnp.float32)
    h = h + jnp.dot(m, w1m_ref[...], preferred_element_type=jnp.float32)
    h = jnp.maximum(h + b1_ref[...], 0.0)
    out_ref[...] = jnp.sum(h * w2_ref[...], axis=1) + b2_ref[0, 0]


def _mlp(u, m, w1u, w1m, b1, w2, b2):
    grid = (BATCH // BS,)
    return pl.pallas_call(
        _mlp_body,
        grid=grid,
        in_specs=[
            pl.BlockSpec((BS, EMBED), lambda i: (i, 0)),
            pl.BlockSpec((BS, EMBED), lambda i: (i, 0)),
            pl.BlockSpec((EMBED, HIDDEN), lambda i: (0, 0)),
            pl.BlockSpec((EMBED, HIDDEN), lambda i: (0, 0)),
            pl.BlockSpec((1, HIDDEN), lambda i: (0, 0)),
            pl.BlockSpec((1, HIDDEN), lambda i: (0, 0)),
            pl.BlockSpec(memory_space=pltpu.SMEM),
        ],
        out_specs=pl.BlockSpec((BS,), lambda i: (i,)),
        out_shape=jax.ShapeDtypeStruct((BATCH,), jnp.float32),
    )(u, m, w1u, w1m, b1, w2, b2)


def kernel(user_ids, movie_ids, user_table, movie_table, W1, b1, W2, b2):
    uids = user_ids.astype(jnp.int32)
    mids = movie_ids.astype(jnp.int32)
    u = _gather_u(user_table, uids)
    m = _gather_m(movie_table, mids)
    w1t = W1.T
    out = _mlp(u, m, w1t[:EMBED], w1t[EMBED:], b1.reshape(1, HIDDEN),
               W2.reshape(1, HIDDEN), b2.reshape(1, 1))
    return out
